# Initial kernel scaffold; baseline (speedup 1.0000x reference)
import jax
import jax.numpy as jnp
from jax import lax
from jax.experimental import pallas as pl
from jax.experimental.pallas import tpu as pltpu
from jax.experimental.pallas import tpu_sc as plsc

N = 10000
E = 320000
D_IN = 128
H = 16
D_OUT = 128

NC = 2
NS = 16
NW = NC * NS
CHUNK = 128
CHUNKS_PER_TILE = -(-E // (CHUNK * NW))
E_PAD = CHUNKS_PER_TILE * CHUNK * NW
EDGES_PER_TILE = CHUNKS_PER_TILE * CHUNK
ROWS_PER_TILE = 628
NPAD = ROWS_PER_TILE * NS

_f32 = jnp.float32


def _fill_rows(ref, n_rows, value):
    row = jnp.full((16,), value, _f32)

    @pl.loop(0, n_rows)
    def _(r):
        ref[r, :] = row


def _hist_body(src_hbm, dst_hbm, outS_hbm, outD_hbm,
               si_v, di_v, ones_v, zbuf_v, accS, accD):
    cid = lax.axis_index("c")
    sid = lax.axis_index("s")
    wid = sid * NC + cid

    _fill_rows(zbuf_v, ROWS_PER_TILE, 0.0)
    _fill_rows(ones_v, CHUNK, 1.0)
    r0 = sid * ROWS_PER_TILE
    pltpu.sync_copy(zbuf_v, accS.at[pl.ds(r0, ROWS_PER_TILE)])
    pltpu.sync_copy(zbuf_v, accD.at[pl.ds(r0, ROWS_PER_TILE)])
    plsc.subcore_barrier()

    base = wid * EDGES_PER_TILE

    @pl.loop(0, CHUNKS_PER_TILE)
    def _(k):
        off = base + k * CHUNK
        pltpu.sync_copy(src_hbm.at[pl.ds(off, CHUNK)], si_v)
        pltpu.sync_copy(dst_hbm.at[pl.ds(off, CHUNK)], di_v)
        pltpu.sync_copy(ones_v, accS.at[si_v], add=True)
        pltpu.sync_copy(ones_v, accD.at[di_v], add=True)

    plsc.subcore_barrier()
    pltpu.sync_copy(accS.at[pl.ds(r0, ROWS_PER_TILE)],
                    outS_hbm.at[cid, pl.ds(r0, ROWS_PER_TILE)])
    pltpu.sync_copy(accD.at[pl.ds(r0, ROWS_PER_TILE)],
                    outD_hbm.at[cid, pl.ds(r0, ROWS_PER_TILE)])


def _agg_body(h_hbm, src_hbm, dst_hbm, out_hbm,
              si_v, di_v, rows_v, zbuf_v, acc):
    cid = lax.axis_index("c")
    sid = lax.axis_index("s")
    wid = sid * NC + cid

    _fill_rows(zbuf_v, ROWS_PER_TILE, 0.0)
    r0 = sid * ROWS_PER_TILE
    pltpu.sync_copy(zbuf_v, acc.at[pl.ds(r0, ROWS_PER_TILE)])
    plsc.subcore_barrier()

    base = wid * EDGES_PER_TILE

    @pl.loop(0, CHUNKS_PER_TILE)
    def _(k):
        off = base + k * CHUNK
        pltpu.sync_copy(src_hbm.at[pl.ds(off, CHUNK)], si_v)
        pltpu.sync_copy(h_hbm.at[si_v], rows_v)
        pltpu.sync_copy(dst_hbm.at[pl.ds(off, CHUNK)], di_v)
        pltpu.sync_copy(rows_v, acc.at[di_v], add=True)

    plsc.subcore_barrier()
    pltpu.sync_copy(acc.at[pl.ds(r0, ROWS_PER_TILE)],
                    out_hbm.at[cid, pl.ds(r0, ROWS_PER_TILE)])


def _make_sc_kernels():
    mesh = plsc.VectorSubcoreMesh(core_axis_name="c", subcore_axis_name="s")
    deg_t = jax.ShapeDtypeStruct((NC, NPAD, H), _f32)
    hist = pl.kernel(
        _hist_body,
        out_type=(deg_t, deg_t),
        mesh=mesh,
        scratch_types=[
            pltpu.VMEM((CHUNK,), jnp.int32),
            pltpu.VMEM((CHUNK,), jnp.int32),
            pltpu.VMEM((CHUNK, H), _f32),
            pltpu.VMEM((ROWS_PER_TILE, H), _f32),
            pltpu.VMEM_SHARED((NPAD, H), _f32),
            pltpu.VMEM_SHARED((NPAD, H), _f32),
        ],
    )
    agg = pl.kernel(
        _agg_body,
        out_type=deg_t,
        mesh=mesh,
        scratch_types=[
            pltpu.VMEM((CHUNK,), jnp.int32),
            pltpu.VMEM((CHUNK,), jnp.int32),
            pltpu.VMEM((CHUNK, H), _f32),
            pltpu.VMEM((ROWS_PER_TILE, H), _f32),
            pltpu.VMEM_SHARED((NPAD, H), _f32),
        ],
    )
    return hist, agg


_HIST, _AGG = _make_sc_kernels()

_BLK = NPAD // 8
_OBLK = N // 8


def _mm1_body(x_ref, w_ref, dS_ref, o_ref):
    deg = dS_ref[0] + dS_ref[1]
    norm = lax.rsqrt(jnp.maximum(deg, 1.0))
    o_ref[...] = jnp.dot(x_ref[...], w_ref[...],
                         preferred_element_type=_f32) * norm


def _mid_body(a_ref, dD_ref, dS_ref, b_ref, o_ref):
    agg = a_ref[0] + a_ref[1]
    nD = lax.rsqrt(jnp.maximum(dD_ref[0] + dD_ref[1], 1.0))
    nS = lax.rsqrt(jnp.maximum(dS_ref[0] + dS_ref[1], 1.0))
    h = jnp.maximum(agg * nD + b_ref[...], 0.0)
    o_ref[...] = h * nS


def _out_body(a_ref, dD_ref, w_ref, b_ref, o_ref):
    agg = a_ref[0] + a_ref[1]
    nD = lax.rsqrt(jnp.maximum(dD_ref[0] + dD_ref[1], 1.0))[:, 0:1]
    o_ref[...] = jnp.dot(agg, w_ref[...],
                         preferred_element_type=_f32) * nD + b_ref[...]


_deg_spec = pl.BlockSpec((NC, _BLK, H), lambda i: (0, i, 0))
_odeg_spec = pl.BlockSpec((NC, _OBLK, H), lambda i: (0, i, 0))

_MM1 = pl.pallas_call(
    _mm1_body,
    grid=(8,),
    in_specs=[
        pl.BlockSpec((_BLK, D_IN), lambda i: (i, 0)),
        pl.BlockSpec((D_IN, H), lambda i: (0, 0)),
        _deg_spec,
    ],
    out_specs=pl.BlockSpec((_BLK, H), lambda i: (i, 0)),
    out_shape=jax.ShapeDtypeStruct((NPAD, H), _f32),
)

_MID = pl.pallas_call(
    _mid_body,
    grid=(8,),
    in_specs=[_deg_spec, _deg_spec, _deg_spec,
              pl.BlockSpec((1, H), lambda i: (0, 0))],
    out_specs=pl.BlockSpec((_BLK, H), lambda i: (i, 0)),
    out_shape=jax.ShapeDtypeStruct((NPAD, H), _f32),
)

_OUT = pl.pallas_call(
    _out_body,
    grid=(8,),
    in_specs=[_odeg_spec, _odeg_spec,
              pl.BlockSpec((H, D_OUT), lambda i: (0, 0)),
              pl.BlockSpec((1, D_OUT), lambda i: (0, 0))],
    out_specs=pl.BlockSpec((_OBLK, D_OUT), lambda i: (i, 0)),
    out_shape=jax.ShapeDtypeStruct((N, D_OUT), _f32),
)


def kernel(node_feat, g, W1, b1, W2, b2):
    src = g[0]
    dst = g[1]
    pad = jnp.full((E_PAD - E,), N, jnp.int32)
    srcp = jnp.concatenate([src, pad])
    dstp = jnp.concatenate([dst, pad])
    x_pad = jnp.pad(node_feat, ((0, NPAD - N), (0, 0)))

    degS, degD = _HIST(srcp, dstp)

    h1s = _MM1(x_pad, W1, degS)
    agg1 = _AGG(h1s, srcp, dstp)
    h2s = _MID(agg1, degD, degS, b1.reshape(1, H))
    agg2 = _AGG(h2s, srcp, dstp)
    out = _OUT(agg2, degD, W2, b2.reshape(1, D_OUT))
    return out


# speedup vs baseline: 5.8231x; 5.8231x over previous
import jax
import jax.numpy as jnp
from jax import lax
from jax.experimental import pallas as pl
from jax.experimental.pallas import tpu as pltpu
from jax.experimental.pallas import tpu_sc as plsc

N = 10000
E = 320000
D_IN = 128
H = 16
D_OUT = 128

NC = 2
NS = 16
NW = NC * NS
CHUNK = 128
CHUNKS_PER_TILE = -(-E // (CHUNK * NW))
E_PAD = CHUNKS_PER_TILE * CHUNK * NW
EDGES_PER_TILE = CHUNKS_PER_TILE * CHUNK
ROWS_PER_TILE = 632
NPAD = ROWS_PER_TILE * NS

_f32 = jnp.float32


def _fill_rows(ref, n_rows, value):
    row = jnp.full((16,), value, _f32)

    @pl.loop(0, n_rows)
    def _(r):
        ref[r, :] = row


def _hist_body(src_hbm, dst_hbm, outS_hbm, outD_hbm,
               si_v, di_v, ones_v, zbuf_v, accS, accD):
    cid = lax.axis_index("c")
    sid = lax.axis_index("s")
    wid = sid * NC + cid

    _fill_rows(zbuf_v, ROWS_PER_TILE, 0.0)
    _fill_rows(ones_v, CHUNK, 1.0)
    r0 = sid * ROWS_PER_TILE
    pltpu.sync_copy(zbuf_v, accS.at[pl.ds(r0, ROWS_PER_TILE)])
    pltpu.sync_copy(zbuf_v, accD.at[pl.ds(r0, ROWS_PER_TILE)])
    plsc.subcore_barrier()

    base = wid * EDGES_PER_TILE

    @pl.loop(0, CHUNKS_PER_TILE)
    def _(k):
        off = base + k * CHUNK
        pltpu.sync_copy(src_hbm.at[pl.ds(off, CHUNK)], si_v)
        pltpu.sync_copy(dst_hbm.at[pl.ds(off, CHUNK)], di_v)
        pltpu.sync_copy(ones_v, accS.at[si_v], add=True)
        pltpu.sync_copy(ones_v, accD.at[di_v], add=True)

    plsc.subcore_barrier()
    pltpu.sync_copy(accS.at[pl.ds(r0, ROWS_PER_TILE)],
                    outS_hbm.at[cid, pl.ds(r0, ROWS_PER_TILE)])
    pltpu.sync_copy(accD.at[pl.ds(r0, ROWS_PER_TILE)],
                    outD_hbm.at[cid, pl.ds(r0, ROWS_PER_TILE)])


def _agg_body(h_hbm, src_hbm, dst_hbm, out_hbm,
              si_v, di_v, rows_v, zbuf_v, acc):
    cid = lax.axis_index("c")
    sid = lax.axis_index("s")
    wid = sid * NC + cid

    _fill_rows(zbuf_v, ROWS_PER_TILE, 0.0)
    r0 = sid * ROWS_PER_TILE
    pltpu.sync_copy(zbuf_v, acc.at[pl.ds(r0, ROWS_PER_TILE)])
    plsc.subcore_barrier()

    base = wid * EDGES_PER_TILE

    @pl.loop(0, CHUNKS_PER_TILE)
    def _(k):
        off = base + k * CHUNK
        pltpu.sync_copy(src_hbm.at[pl.ds(off, CHUNK)], si_v)
        pltpu.sync_copy(h_hbm.at[si_v], rows_v)
        pltpu.sync_copy(dst_hbm.at[pl.ds(off, CHUNK)], di_v)
        pltpu.sync_copy(rows_v, acc.at[di_v], add=True)

    plsc.subcore_barrier()
    pltpu.sync_copy(acc.at[pl.ds(r0, ROWS_PER_TILE)],
                    out_hbm.at[cid, pl.ds(r0, ROWS_PER_TILE)])


def _make_sc_kernels():
    mesh = plsc.VectorSubcoreMesh(core_axis_name="c", subcore_axis_name="s")
    cp = pltpu.CompilerParams(use_tc_tiling_on_sc=False)
    deg_t = jax.ShapeDtypeStruct((NC, NPAD, H), _f32)
    hist = pl.kernel(
        _hist_body,
        out_type=(deg_t, deg_t),
        mesh=mesh,
        compiler_params=cp,
        scratch_types=[
            pltpu.VMEM((CHUNK,), jnp.int32),
            pltpu.VMEM((CHUNK,), jnp.int32),
            pltpu.VMEM((CHUNK, H), _f32),
            pltpu.VMEM((ROWS_PER_TILE, H), _f32),
            pltpu.VMEM_SHARED((NPAD, H), _f32),
            pltpu.VMEM_SHARED((NPAD, H), _f32),
        ],
    )
    agg = pl.kernel(
        _agg_body,
        out_type=deg_t,
        mesh=mesh,
        compiler_params=cp,
        scratch_types=[
            pltpu.VMEM((CHUNK,), jnp.int32),
            pltpu.VMEM((CHUNK,), jnp.int32),
            pltpu.VMEM((CHUNK, H), _f32),
            pltpu.VMEM((ROWS_PER_TILE, H), _f32),
            pltpu.VMEM_SHARED((NPAD, H), _f32),
        ],
    )
    return hist, agg


_HIST, _AGG = _make_sc_kernels()

_BLK = NPAD // 8


def _mm1_body(x_ref, w_ref, dS_ref, o_ref):
    deg = dS_ref[0] + dS_ref[1]
    norm = lax.rsqrt(jnp.maximum(deg, 1.0))
    o_ref[...] = jnp.dot(x_ref[...], w_ref[...],
                         preferred_element_type=_f32) * norm


def _mid_body(a_ref, dD_ref, dS_ref, b_ref, o_ref):
    agg = a_ref[0] + a_ref[1]
    nD = lax.rsqrt(jnp.maximum(dD_ref[0] + dD_ref[1], 1.0))
    nS = lax.rsqrt(jnp.maximum(dS_ref[0] + dS_ref[1], 1.0))
    h = jnp.maximum(agg * nD + b_ref[...], 0.0)
    o_ref[...] = h * nS


def _out_body(a_ref, dD_ref, w_ref, b_ref, o_ref):
    agg = a_ref[0] + a_ref[1]
    nD = lax.rsqrt(jnp.maximum(dD_ref[0] + dD_ref[1], 1.0))[:, 0:1]
    o_ref[...] = jnp.dot(agg, w_ref[...],
                         preferred_element_type=_f32) * nD + b_ref[...]


_deg_spec = pl.BlockSpec((NC, _BLK, H), lambda i: (0, i, 0))

_MM1 = pl.pallas_call(
    _mm1_body,
    grid=(8,),
    in_specs=[
        pl.BlockSpec((_BLK, D_IN), lambda i: (i, 0)),
        pl.BlockSpec((D_IN, H), lambda i: (0, 0)),
        _deg_spec,
    ],
    out_specs=pl.BlockSpec((_BLK, H), lambda i: (i, 0)),
    out_shape=jax.ShapeDtypeStruct((NPAD, H), _f32),
)

_MID = pl.pallas_call(
    _mid_body,
    grid=(8,),
    in_specs=[_deg_spec, _deg_spec, _deg_spec,
              pl.BlockSpec((1, H), lambda i: (0, 0))],
    out_specs=pl.BlockSpec((_BLK, H), lambda i: (i, 0)),
    out_shape=jax.ShapeDtypeStruct((NPAD, H), _f32),
)

_OUT = pl.pallas_call(
    _out_body,
    grid=(8,),
    in_specs=[_deg_spec, _deg_spec,
              pl.BlockSpec((H, D_OUT), lambda i: (0, 0)),
              pl.BlockSpec((1, D_OUT), lambda i: (0, 0))],
    out_specs=pl.BlockSpec((_BLK, D_OUT), lambda i: (i, 0)),
    out_shape=jax.ShapeDtypeStruct((NPAD, D_OUT), _f32),
)


def kernel(node_feat, g, W1, b1, W2, b2):
    src = g[0]
    dst = g[1]
    pad = jnp.full((E_PAD - E,), N, jnp.int32)
    srcp = jnp.concatenate([src, pad])
    dstp = jnp.concatenate([dst, pad])
    x_pad = jnp.pad(node_feat, ((0, NPAD - N), (0, 0)))

    degS, degD = _HIST(srcp, dstp)

    h1s = _MM1(x_pad, W1, degS)
    agg1 = _AGG(h1s, srcp, dstp)
    h2s = _MID(agg1, degD, degS, b1.reshape(1, H))
    agg2 = _AGG(h2s, srcp, dstp)
    out = _OUT(agg2, degD, W2, b2.reshape(1, D_OUT))
    return out[:N]


# speedup vs baseline: 10.6897x; 1.8357x over previous
import jax
import jax.numpy as jnp
from jax import lax
from jax.experimental import pallas as pl
from jax.experimental.pallas import tpu as pltpu
from jax.experimental.pallas import tpu_sc as plsc

N = 10000
E = 320000
D_IN = 128
H = 16
D_OUT = 128

NC = 2
NS = 16
NW = NC * NS
CHUNK = 128
NBUF = 4
CHUNKS_PER_TILE = 80
GROUPS = CHUNKS_PER_TILE // NBUF
E_PAD = CHUNKS_PER_TILE * CHUNK * NW
EDGES_PER_TILE = CHUNKS_PER_TILE * CHUNK
ROWS_PER_TILE = 632
NPAD = ROWS_PER_TILE * NS

_f32 = jnp.float32


def _fill_rows(ref, n_rows, value):
    row = jnp.full((16,), value, _f32)

    @pl.loop(0, n_rows)
    def _(r):
        ref[r, :] = row


def _hist_body(src_hbm, dst_hbm, outS_hbm, outD_hbm,
               si_all, di_all, ones_v, zbuf_v, accS, accD, semA, semB):
    cid = lax.axis_index("c")
    sid = lax.axis_index("s")
    wid = sid * NC + cid

    _fill_rows(zbuf_v, ROWS_PER_TILE, 0.0)
    _fill_rows(ones_v, CHUNK, 1.0)
    r0 = sid * ROWS_PER_TILE
    pltpu.sync_copy(src_hbm.at[wid], si_all)
    pltpu.sync_copy(dst_hbm.at[wid], di_all)
    pltpu.sync_copy(zbuf_v, accS.at[pl.ds(r0, ROWS_PER_TILE)])
    pltpu.sync_copy(zbuf_v, accD.at[pl.ds(r0, ROWS_PER_TILE)])
    plsc.subcore_barrier()

    @pl.loop(0, CHUNKS_PER_TILE)
    def _(k):
        pltpu.async_copy(ones_v, accS.at[si_all.at[k]], semA, add=True)
        pltpu.async_copy(ones_v, accD.at[di_all.at[k]], semB, add=True)

        @pl.when(k >= NBUF)
        def _():
            pltpu.make_async_copy(ones_v, accS.at[si_all.at[0]], semA).wait()
            pltpu.make_async_copy(ones_v, accD.at[di_all.at[0]], semB).wait()

    for _ in range(NBUF):
        pltpu.make_async_copy(ones_v, accS.at[si_all.at[0]], semA).wait()
        pltpu.make_async_copy(ones_v, accD.at[di_all.at[0]], semB).wait()

    plsc.subcore_barrier()
    pltpu.sync_copy(accS.at[pl.ds(r0, ROWS_PER_TILE)],
                    outS_hbm.at[cid, pl.ds(r0, ROWS_PER_TILE)])
    pltpu.sync_copy(accD.at[pl.ds(r0, ROWS_PER_TILE)],
                    outD_hbm.at[cid, pl.ds(r0, ROWS_PER_TILE)])


def _agg_body(h_hbm, src_hbm, dst_hbm, out_hbm,
              si_all, di_all, rows, zbuf_v, acc, sg, ss):
    cid = lax.axis_index("c")
    sid = lax.axis_index("s")
    wid = sid * NC + cid

    _fill_rows(zbuf_v, ROWS_PER_TILE, 0.0)
    r0 = sid * ROWS_PER_TILE
    pltpu.sync_copy(src_hbm.at[wid], si_all)
    pltpu.sync_copy(dst_hbm.at[wid], di_all)
    pltpu.sync_copy(zbuf_v, acc.at[pl.ds(r0, ROWS_PER_TILE)])
    plsc.subcore_barrier()

    def wait_gather(b):
        pltpu.make_async_copy(h_hbm.at[si_all.at[0]], rows[b], sg[b]).wait()

    def wait_scatter(b):
        pltpu.make_async_copy(rows[b], acc.at[di_all.at[0]], ss[b]).wait()

    for b in range(NBUF):
        pltpu.async_copy(h_hbm.at[si_all.at[b]], rows[b], sg[b])

    @pl.loop(0, GROUPS - 1)
    def _(g):
        k0 = g * NBUF
        for b in range(NBUF):
            wait_gather(b)
            pltpu.async_copy(rows[b], acc.at[di_all.at[k0 + b]], ss[b],
                             add=True)
        for b in range(NBUF):
            wait_scatter(b)
            pltpu.async_copy(h_hbm.at[si_all.at[k0 + NBUF + b]], rows[b],
                             sg[b])

    k0 = (GROUPS - 1) * NBUF
    for b in range(NBUF):
        wait_gather(b)
        pltpu.async_copy(rows[b], acc.at[di_all.at[k0 + b]], ss[b], add=True)
    for b in range(NBUF):
        wait_scatter(b)

    plsc.subcore_barrier()
    pltpu.sync_copy(acc.at[pl.ds(r0, ROWS_PER_TILE)],
                    out_hbm.at[cid, pl.ds(r0, ROWS_PER_TILE)])


def _make_sc_kernels():
    mesh = plsc.VectorSubcoreMesh(core_axis_name="c", subcore_axis_name="s")
    cp = pltpu.CompilerParams(use_tc_tiling_on_sc=False)
    deg_t = jax.ShapeDtypeStruct((NC, NPAD, H), _f32)
    idx_t = pltpu.VMEM((CHUNKS_PER_TILE, CHUNK), jnp.int32)
    hist = pl.kernel(
        _hist_body,
        out_type=(deg_t, deg_t),
        mesh=mesh,
        compiler_params=cp,
        scratch_types=[
            idx_t,
            idx_t,
            pltpu.VMEM((CHUNK, H), _f32),
            pltpu.VMEM((ROWS_PER_TILE, H), _f32),
            pltpu.VMEM_SHARED((NPAD, H), _f32),
            pltpu.VMEM_SHARED((NPAD, H), _f32),
            pltpu.SemaphoreType.DMA,
            pltpu.SemaphoreType.DMA,
        ],
    )
    agg = pl.kernel(
        _agg_body,
        out_type=deg_t,
        mesh=mesh,
        compiler_params=cp,
        scratch_types=[
            idx_t,
            idx_t,
            [pltpu.VMEM((CHUNK, H), _f32) for _ in range(NBUF)],
            pltpu.VMEM((ROWS_PER_TILE, H), _f32),
            pltpu.VMEM_SHARED((NPAD, H), _f32),
            [pltpu.SemaphoreType.DMA for _ in range(NBUF)],
            [pltpu.SemaphoreType.DMA for _ in range(NBUF)],
        ],
    )
    return hist, agg


_HIST, _AGG = _make_sc_kernels()

_BLK = NPAD // 8


def _mm1_body(x_ref, w_ref, dS_ref, o_ref):
    deg = dS_ref[0] + dS_ref[1]
    norm = lax.rsqrt(jnp.maximum(deg, 1.0))
    o_ref[...] = jnp.dot(x_ref[...], w_ref[...],
                         preferred_element_type=_f32) * norm


def _mid_body(a_ref, dD_ref, dS_ref, b_ref, o_ref):
    agg = a_ref[0] + a_ref[1]
    nD = lax.rsqrt(jnp.maximum(dD_ref[0] + dD_ref[1], 1.0))
    nS = lax.rsqrt(jnp.maximum(dS_ref[0] + dS_ref[1], 1.0))
    h = jnp.maximum(agg * nD + b_ref[...], 0.0)
    o_ref[...] = h * nS


def _out_body(a_ref, dD_ref, w_ref, b_ref, o_ref):
    agg = a_ref[0] + a_ref[1]
    nD = lax.rsqrt(jnp.maximum(dD_ref[0] + dD_ref[1], 1.0))[:, 0:1]
    o_ref[...] = jnp.dot(agg, w_ref[...],
                         preferred_element_type=_f32) * nD + b_ref[...]


_deg_spec = pl.BlockSpec((NC, _BLK, H), lambda i: (0, i, 0))

_MM1 = pl.pallas_call(
    _mm1_body,
    grid=(8,),
    in_specs=[
        pl.BlockSpec((_BLK, D_IN), lambda i: (i, 0)),
        pl.BlockSpec((D_IN, H), lambda i: (0, 0)),
        _deg_spec,
    ],
    out_specs=pl.BlockSpec((_BLK, H), lambda i: (i, 0)),
    out_shape=jax.ShapeDtypeStruct((NPAD, H), _f32),
)

_MID = pl.pallas_call(
    _mid_body,
    grid=(8,),
    in_specs=[_deg_spec, _deg_spec, _deg_spec,
              pl.BlockSpec((1, H), lambda i: (0, 0))],
    out_specs=pl.BlockSpec((_BLK, H), lambda i: (i, 0)),
    out_shape=jax.ShapeDtypeStruct((NPAD, H), _f32),
)

_OUT = pl.pallas_call(
    _out_body,
    grid=(8,),
    in_specs=[_deg_spec, _deg_spec,
              pl.BlockSpec((H, D_OUT), lambda i: (0, 0)),
              pl.BlockSpec((1, D_OUT), lambda i: (0, 0))],
    out_specs=pl.BlockSpec((_BLK, D_OUT), lambda i: (i, 0)),
    out_shape=jax.ShapeDtypeStruct((NPAD, D_OUT), _f32),
)


def kernel(node_feat, g, W1, b1, W2, b2):
    src = g[0]
    dst = g[1]
    pad = jnp.full((E_PAD - E,), N, jnp.int32)
    srcp = jnp.concatenate([src, pad]).reshape(NW, CHUNKS_PER_TILE, CHUNK)
    dstp = jnp.concatenate([dst, pad]).reshape(NW, CHUNKS_PER_TILE, CHUNK)
    x_pad = jnp.pad(node_feat, ((0, NPAD - N), (0, 0)))

    degS, degD = _HIST(srcp, dstp)

    h1s = _MM1(x_pad, W1, degS)
    agg1 = _AGG(h1s, srcp, dstp)
    h2s = _MID(agg1, degD, degS, b1.reshape(1, H))
    agg2 = _AGG(h2s, srcp, dstp)
    out = _OUT(agg2, degD, W2, b2.reshape(1, D_OUT))
    return out[:N]
